# baseline (speedup 1.0000x reference)
import functools
import math

import jax
import jax.numpy as jnp
from jax.experimental import pallas as pl
from jax.experimental.pallas import tpu as pltpu

_RATIO = 0.5
_K1_CHUNK = 2944
_K2_BLK = 800
_NOEDGE = jnp.int32(0x40000000)


def _skey(f32val):
    bits = jax.lax.bitcast_convert_type(f32val, jnp.uint32)
    sign = (bits >> jnp.uint32(31)).astype(jnp.bool_)
    ukey = jnp.where(sign, ~bits, bits | jnp.uint32(0x80000000))
    return jax.lax.bitcast_convert_type(ukey ^ jnp.uint32(0x80000000),
                                        jnp.int32)


def _k1_body(inc_ref, x_ref, xe_ref, dv_ref, inc8_ref, acc, deacc,
             *, nprog, rem, nblk, e):
    pi = pl.program_id(0)
    inc = inc_ref[...]
    x = x_ref[...]
    inc8_ref[...] = inc.astype(jnp.int8)
    if rem != nblk:
        ridx = jax.lax.broadcasted_iota(jnp.int32, (nblk, 1), 0)
        valid = ridx < jnp.where(pi == nprog - 1, rem, nblk)
        inc = jnp.where(valid, inc, 0.0)
        x = jnp.where(valid, x, 0.0)
    part = jax.lax.dot_general(inc, x, (((0,), (0,)), ((), ())),
                               preferred_element_type=jnp.float32)
    desub = jnp.sum(inc, axis=0, keepdims=True)

    @pl.when(pi == 0)
    def _():
        acc[...] = part
        deacc[...] = desub

    @pl.when(pi > 0)
    def _():
        acc[...] += part
        deacc[...] += desub

    dv_ref[...] = jnp.sum(inc, axis=1, keepdims=True)

    @pl.when(pi == nprog - 1)
    def _():
        de = jnp.maximum(deacc[...], 1.0)
        xe_ref[...] = (acc[...] / de.reshape(e, 1)).astype(jnp.bfloat16)


def _stage1(node_features, incidence):
    n, d = node_features.shape
    e = incidence.shape[1]
    nblk = _K1_CHUNK
    nprog = -(-n // nblk)
    rem = n - (nprog - 1) * nblk
    xe, dv, inc8 = pl.pallas_call(
        functools.partial(_k1_body, nprog=nprog, rem=rem, nblk=nblk, e=e),
        grid=(nprog,),
        in_specs=[
            pl.BlockSpec((nblk, e), lambda i: (i, 0)),
            pl.BlockSpec((nblk, d), lambda i: (i, 0)),
        ],
        out_specs=[
            pl.BlockSpec((e, d), lambda i: (0, 0)),
            pl.BlockSpec((nblk, 1), lambda i: (i, 0)),
            pl.BlockSpec((nblk, e), lambda i: (i, 0)),
        ],
        out_shape=[
            jax.ShapeDtypeStruct((e, d), jnp.bfloat16),
            jax.ShapeDtypeStruct((nprog * nblk, 1), jnp.float32),
            jax.ShapeDtypeStruct((nprog * nblk, e), jnp.int8),
        ],
        scratch_shapes=[pltpu.VMEM((e, d), jnp.float32),
                        pltpu.VMEM((1, e), jnp.float32)],
    )(incidence, node_features)
    return xe, dv[:n], inc8


def _k2_body(inc_ref, xe_ref, dv_ref, x_ref, nm_ref, w_ref, b_ref, p_ref,
             sc_ref, gated_ref, mk_ref, mi_ref, mks, mis,
             *, nblk, nprog):
    pi = pl.program_id(0)
    inc8 = inc_ref[...]
    incb = inc8.astype(jnp.bfloat16)
    rawT = jax.lax.dot_general(
        xe_ref[...], incb,
        (((0,), (1,)), ((), ())), preferred_element_type=jnp.float32)
    xvT = (rawT / jnp.maximum(dv_ref[...], 1.0).T).astype(jnp.bfloat16)
    emb = (jax.lax.dot_general(
        xvT.astype(jnp.float32), w_ref[...], (((0,), (0,)), ((), ())),
        preferred_element_type=jnp.float32) + b_ref[...]).astype(jnp.bfloat16)
    logits = jnp.dot(emb.astype(jnp.float32), p_ref[...],
                     preferred_element_type=jnp.float32)
    score = 1.0 / (1.0 + jnp.exp(-logits))
    score = jnp.where(nm_ref[...] > 0, score, -jnp.inf)
    sc_ref[...] = score
    gated_ref[...] = x_ref[...] * score

    member = inc8.astype(jnp.float32) > 0.0
    keys = _skey(score)
    idx = (pi * nblk
           + jax.lax.broadcasted_iota(jnp.int32, (nblk, 1), 0))
    kne = jnp.where(member, keys, -2147483648).astype(jnp.int32)
    mk_blk = jnp.max(kne, axis=0, keepdims=True)
    hit = kne == mk_blk
    ine = jnp.where(hit, idx, _NOEDGE).astype(jnp.int32)
    mi_blk = jnp.min(ine, axis=0, keepdims=True)

    @pl.when(pi == 0)
    def _():
        mks[...] = mk_blk
        mis[...] = mi_blk

    @pl.when(pi > 0)
    def _():
        better = mk_blk > mks[...]
        same = mk_blk == mks[...]
        mis[...] = jnp.where(better, mi_blk,
                             jnp.where(same, jnp.minimum(mis[...], mi_blk),
                                       mis[...]))
        mks[...] = jnp.maximum(mks[...], mk_blk)

    @pl.when(pi == nprog - 1)
    def _():
        mk_ref[...] = mks[...]
        mi_ref[...] = mis[...]


def _stage2(inc8, xe, dv, node_features, node_mask_f, W, b_row, proj_col):
    n, d = node_features.shape
    e = inc8.shape[1]
    nblk = _K2_BLK
    nprog = n // nblk
    return pl.pallas_call(
        functools.partial(_k2_body, nblk=nblk, nprog=nprog),
        grid=(nprog,),
        in_specs=[
            pl.BlockSpec((nblk, e), lambda i: (i, 0)),
            pl.BlockSpec((e, d), lambda i: (0, 0)),
            pl.BlockSpec((nblk, 1), lambda i: (i, 0)),
            pl.BlockSpec((nblk, d), lambda i: (i, 0)),
            pl.BlockSpec((nblk, 1), lambda i: (i, 0)),
            pl.BlockSpec((d, d), lambda i: (0, 0)),
            pl.BlockSpec((1, d), lambda i: (0, 0)),
            pl.BlockSpec((d, 1), lambda i: (0, 0)),
        ],
        out_specs=[
            pl.BlockSpec((nblk, 1), lambda i: (i, 0)),
            pl.BlockSpec((nblk, d), lambda i: (i, 0)),
            pl.BlockSpec((1, e), lambda i: (0, 0)),
            pl.BlockSpec((1, e), lambda i: (0, 0)),
        ],
        out_shape=[
            jax.ShapeDtypeStruct((n, 1), jnp.float32),
            jax.ShapeDtypeStruct((n, d), jnp.float32),
            jax.ShapeDtypeStruct((1, e), jnp.int32),
            jax.ShapeDtypeStruct((1, e), jnp.int32),
        ],
        scratch_shapes=[pltpu.VMEM((1, e), jnp.int32),
                        pltpu.VMEM((1, e), jnp.int32)],
    )(inc8, xe, dv, node_features, node_mask_f, W, b_row, proj_col)


def _k3_body(s_ref, mk_ref, mi_ref, em_ref, mask_ref, emask_ref,
             *, n, k, rows):
    s = s_ref[...]
    bits = jax.lax.bitcast_convert_type(s, jnp.uint32)
    sign = (bits >> jnp.uint32(31)).astype(jnp.bool_)
    keys = jnp.where(sign, ~bits, bits | jnp.uint32(0x80000000))
    ridx = jax.lax.broadcasted_iota(jnp.int32, (rows, 128), 0)
    lidx = jax.lax.broadcasted_iota(jnp.int32, (rows, 128), 1)
    flat = ridx * 128 + lidx
    keys = jnp.where(flat < n, keys, jnp.uint32(0))

    def body(i, p):
        test = p | (jnp.uint32(1) << (jnp.uint32(31) - i.astype(jnp.uint32)))
        cnt = jnp.sum((keys >= test).astype(jnp.int32))
        return jnp.where(cnt >= k, test, p)

    t_key = jax.lax.fori_loop(0, 32, body, jnp.uint32(0))
    greater = jnp.sum((keys > t_key).astype(jnp.int32))
    need = (k - greater).astype(jnp.float32)

    tie = keys == t_key
    tf = tie.astype(jnp.float32)
    li = jax.lax.broadcasted_iota(jnp.int32, (128, 128), 0)
    lj = jax.lax.broadcasted_iota(jnp.int32, (128, 128), 1)
    lane_lt = (li < lj).astype(jnp.bfloat16)
    rowcum = jnp.dot(tf.astype(jnp.bfloat16), lane_lt,
                     preferred_element_type=jnp.float32)
    rs = jnp.sum(tf, axis=1, keepdims=True)
    ri = jax.lax.broadcasted_iota(jnp.int32, (rows, rows), 0)
    rj = jax.lax.broadcasted_iota(jnp.int32, (rows, rows), 1)
    row_lt = (rj < ri).astype(jnp.bfloat16)
    offs = jnp.dot(row_lt, rs.astype(jnp.bfloat16),
                   preferred_element_type=jnp.float32)
    rank = rowcum + offs
    sel = tie & (rank < need)
    mask_ref[...] = (keys > t_key) | sel

    st = jax.lax.bitcast_convert_type(t_key ^ jnp.uint32(0x80000000),
                                      jnp.int32)
    cutoff = jnp.max(jnp.where(sel, flat, jnp.int32(-1)))
    mk = mk_ref[...]
    mi = mi_ref[...]
    active = (mk > st) | ((mk == st) & (mi <= cutoff))
    emask_ref[...] = active & (em_ref[...] > 0)


def _stage3(scores_col, mk, mi, edge_mask_f, n, k):
    e = mk.shape[1]
    rows = (n + 127) // 128
    rows = ((rows + 7) // 8) * 8
    total = rows * 128
    spad = jnp.pad(scores_col[:, 0], (0, total - n),
                   constant_values=-jnp.inf).reshape(rows, 128)
    mask2d, emask = pl.pallas_call(
        functools.partial(_k3_body, n=n, k=k, rows=rows),
        out_shape=[jax.ShapeDtypeStruct((rows, 128), jnp.bool_),
                   jax.ShapeDtypeStruct((1, e), jnp.bool_)],
    )(spad, mk, mi, edge_mask_f)
    return mask2d.reshape(total)[:n], emask[0]


def kernel(node_features, incidence, node_mask, edge_mask, W, b, proj):
    n, d = node_features.shape
    e = incidence.shape[1]
    k = max(1, math.ceil(_RATIO * n))

    xe, dv, inc8 = _stage1(node_features, incidence)
    scores, gated, mk, mi = _stage2(
        inc8, xe, dv, node_features,
        node_mask.astype(jnp.float32).reshape(n, 1),
        W, b.reshape(1, d), proj.reshape(d, 1))
    node_mask_out, edge_mask_out = _stage3(
        scores, mk, mi, edge_mask.astype(jnp.float32).reshape(1, e), n, k)
    return gated, node_mask_out, edge_mask_out


# speedup vs baseline: 1.1048x; 1.1048x over previous
import functools
import math

import jax
import jax.numpy as jnp
from jax.experimental import pallas as pl
from jax.experimental.pallas import tpu as pltpu

_RATIO = 0.5
_K1_CHUNK = 2944
_K2_BLK = 2000
_NOEDGE = jnp.int32(0x40000000)


def _skey(f32val):
    bits = jax.lax.bitcast_convert_type(f32val, jnp.uint32)
    sign = (bits >> jnp.uint32(31)).astype(jnp.bool_)
    ukey = jnp.where(sign, ~bits, bits | jnp.uint32(0x80000000))
    return jax.lax.bitcast_convert_type(ukey ^ jnp.uint32(0x80000000),
                                        jnp.int32)


def _k1_body(inc_ref, x_ref, xe_ref, dv_ref, acc, deacc,
             *, nprog, rem, nblk, e):
    pi = pl.program_id(0)
    inc = inc_ref[...]
    x = x_ref[...]
    if rem != nblk:
        ridx = jax.lax.broadcasted_iota(jnp.int32, (nblk, 1), 0)
        valid = ridx < jnp.where(pi == nprog - 1, rem, nblk)
        inc = jnp.where(valid, inc, 0.0)
        x = jnp.where(valid, x, 0.0)
    part = jax.lax.dot_general(inc, x, (((0,), (0,)), ((), ())),
                               preferred_element_type=jnp.float32)
    desub = jnp.sum(inc, axis=0, keepdims=True)

    @pl.when(pi == 0)
    def _():
        acc[...] = part
        deacc[...] = desub

    @pl.when(pi > 0)
    def _():
        acc[...] += part
        deacc[...] += desub

    dv_ref[...] = jnp.sum(inc, axis=1, keepdims=True)

    @pl.when(pi == nprog - 1)
    def _():
        de = jnp.maximum(deacc[...], 1.0)
        xe_ref[...] = (acc[...] / de.reshape(e, 1)).astype(jnp.bfloat16)


def _stage1(node_features, incidence):
    n, d = node_features.shape
    e = incidence.shape[1]
    nblk = _K1_CHUNK
    nprog = -(-n // nblk)
    rem = n - (nprog - 1) * nblk
    xe, dv = pl.pallas_call(
        functools.partial(_k1_body, nprog=nprog, rem=rem, nblk=nblk, e=e),
        grid=(nprog,),
        in_specs=[
            pl.BlockSpec((nblk, e), lambda i: (i, 0)),
            pl.BlockSpec((nblk, d), lambda i: (i, 0)),
        ],
        out_specs=[
            pl.BlockSpec((e, d), lambda i: (0, 0)),
            pl.BlockSpec((nblk, 1), lambda i: (i, 0)),
        ],
        out_shape=[
            jax.ShapeDtypeStruct((e, d), jnp.bfloat16),
            jax.ShapeDtypeStruct((nprog * nblk, 1), jnp.float32),
        ],
        scratch_shapes=[pltpu.VMEM((e, d), jnp.float32),
                        pltpu.VMEM((1, e), jnp.float32)],
    )(incidence, node_features)
    return xe, dv[:n]


def _k2_body(inc_ref, xe_ref, dv_ref, x_ref, nm_ref, w_ref, b_ref, p_ref,
             sc_ref, gated_ref, mk_ref, mi_ref, mks, mis,
             *, nblk, nprog):
    pi = pl.program_id(0)
    inc = inc_ref[...]
    rawT = jax.lax.dot_general(
        xe_ref[...].astype(jnp.float32), inc,
        (((0,), (1,)), ((), ())), preferred_element_type=jnp.float32)
    xvT = (rawT / jnp.maximum(dv_ref[...], 1.0).T).astype(jnp.bfloat16)
    emb = (jax.lax.dot_general(
        xvT.astype(jnp.float32), w_ref[...], (((0,), (0,)), ((), ())),
        preferred_element_type=jnp.float32) + b_ref[...]).astype(jnp.bfloat16)
    logits = jnp.dot(emb.astype(jnp.float32), p_ref[...],
                     preferred_element_type=jnp.float32)
    score = 1.0 / (1.0 + jnp.exp(-logits))
    score = jnp.where(nm_ref[...] > 0, score, -jnp.inf)
    sc_ref[...] = score
    gated_ref[...] = x_ref[...] * score

    member = inc > 0.0
    keys = _skey(score)
    idx = (pi * nblk
           + jax.lax.broadcasted_iota(jnp.int32, (nblk, 1), 0))
    kne = jnp.where(member, keys, -2147483648).astype(jnp.int32)
    mk_blk = jnp.max(kne, axis=0, keepdims=True)
    hit = kne == mk_blk
    ine = jnp.where(hit, idx, _NOEDGE).astype(jnp.int32)
    mi_blk = jnp.min(ine, axis=0, keepdims=True)

    @pl.when(pi == 0)
    def _():
        mks[...] = mk_blk
        mis[...] = mi_blk

    @pl.when(pi > 0)
    def _():
        better = mk_blk > mks[...]
        same = mk_blk == mks[...]
        mis[...] = jnp.where(better, mi_blk,
                             jnp.where(same, jnp.minimum(mis[...], mi_blk),
                                       mis[...]))
        mks[...] = jnp.maximum(mks[...], mk_blk)

    @pl.when(pi == nprog - 1)
    def _():
        mk_ref[...] = mks[...]
        mi_ref[...] = mis[...]


def _stage2(incidence, xe, dv, node_features, node_mask_f, W, b_row, proj_col):
    n, d = node_features.shape
    e = incidence.shape[1]
    nblk = _K2_BLK
    nprog = n // nblk
    return pl.pallas_call(
        functools.partial(_k2_body, nblk=nblk, nprog=nprog),
        grid=(nprog,),
        in_specs=[
            pl.BlockSpec((nblk, e), lambda i: (i, 0)),
            pl.BlockSpec((e, d), lambda i: (0, 0)),
            pl.BlockSpec((nblk, 1), lambda i: (i, 0)),
            pl.BlockSpec((nblk, d), lambda i: (i, 0)),
            pl.BlockSpec((nblk, 1), lambda i: (i, 0)),
            pl.BlockSpec((d, d), lambda i: (0, 0)),
            pl.BlockSpec((1, d), lambda i: (0, 0)),
            pl.BlockSpec((d, 1), lambda i: (0, 0)),
        ],
        out_specs=[
            pl.BlockSpec((nblk, 1), lambda i: (i, 0)),
            pl.BlockSpec((nblk, d), lambda i: (i, 0)),
            pl.BlockSpec((1, e), lambda i: (0, 0)),
            pl.BlockSpec((1, e), lambda i: (0, 0)),
        ],
        out_shape=[
            jax.ShapeDtypeStruct((n, 1), jnp.float32),
            jax.ShapeDtypeStruct((n, d), jnp.float32),
            jax.ShapeDtypeStruct((1, e), jnp.int32),
            jax.ShapeDtypeStruct((1, e), jnp.int32),
        ],
        scratch_shapes=[pltpu.VMEM((1, e), jnp.int32),
                        pltpu.VMEM((1, e), jnp.int32)],
    )(incidence, xe, dv, node_features, node_mask_f, W, b_row, proj_col)


def _k3_body(s_ref, mk_ref, mi_ref, em_ref, mask_ref, emask_ref,
             *, n, k, rows):
    s = s_ref[...]
    bits = jax.lax.bitcast_convert_type(s, jnp.uint32)
    sign = (bits >> jnp.uint32(31)).astype(jnp.bool_)
    keys = jnp.where(sign, ~bits, bits | jnp.uint32(0x80000000))
    ridx = jax.lax.broadcasted_iota(jnp.int32, (rows, 128), 0)
    lidx = jax.lax.broadcasted_iota(jnp.int32, (rows, 128), 1)
    flat = ridx * 128 + lidx
    keys = jnp.where(flat < n, keys, jnp.uint32(0))

    def body(i, p):
        test = p | (jnp.uint32(1) << (jnp.uint32(31) - i.astype(jnp.uint32)))
        cnt = jnp.sum((keys >= test).astype(jnp.int32))
        return jnp.where(cnt >= k, test, p)

    t_key = jax.lax.fori_loop(0, 32, body, jnp.uint32(0))
    greater = jnp.sum((keys > t_key).astype(jnp.int32))
    need = (k - greater).astype(jnp.float32)

    tie = keys == t_key
    tf = tie.astype(jnp.float32)
    li = jax.lax.broadcasted_iota(jnp.int32, (128, 128), 0)
    lj = jax.lax.broadcasted_iota(jnp.int32, (128, 128), 1)
    lane_lt = (li < lj).astype(jnp.bfloat16)
    rowcum = jnp.dot(tf.astype(jnp.bfloat16), lane_lt,
                     preferred_element_type=jnp.float32)
    rs = jnp.sum(tf, axis=1, keepdims=True)
    ri = jax.lax.broadcasted_iota(jnp.int32, (rows, rows), 0)
    rj = jax.lax.broadcasted_iota(jnp.int32, (rows, rows), 1)
    row_lt = (rj < ri).astype(jnp.bfloat16)
    offs = jnp.dot(row_lt, rs.astype(jnp.bfloat16),
                   preferred_element_type=jnp.float32)
    rank = rowcum + offs
    sel = tie & (rank < need)
    mask_ref[...] = (keys > t_key) | sel

    st = jax.lax.bitcast_convert_type(t_key ^ jnp.uint32(0x80000000),
                                      jnp.int32)
    cutoff = jnp.max(jnp.where(sel, flat, jnp.int32(-1)))
    mk = mk_ref[...]
    mi = mi_ref[...]
    active = (mk > st) | ((mk == st) & (mi <= cutoff))
    emask_ref[...] = active & (em_ref[...] > 0)


def _stage3(scores_col, mk, mi, edge_mask_f, n, k):
    e = mk.shape[1]
    rows = (n + 127) // 128
    rows = ((rows + 7) // 8) * 8
    total = rows * 128
    spad = jnp.pad(scores_col[:, 0], (0, total - n),
                   constant_values=-jnp.inf).reshape(rows, 128)
    mask2d, emask = pl.pallas_call(
        functools.partial(_k3_body, n=n, k=k, rows=rows),
        out_shape=[jax.ShapeDtypeStruct((rows, 128), jnp.bool_),
                   jax.ShapeDtypeStruct((1, e), jnp.bool_)],
    )(spad, mk, mi, edge_mask_f)
    return mask2d.reshape(total)[:n], emask[0]


def kernel(node_features, incidence, node_mask, edge_mask, W, b, proj):
    n, d = node_features.shape
    e = incidence.shape[1]
    k = max(1, math.ceil(_RATIO * n))

    xe, dv = _stage1(node_features, incidence)
    scores, gated, mk, mi = _stage2(
        incidence, xe, dv, node_features,
        node_mask.astype(jnp.float32).reshape(n, 1),
        W, b.reshape(1, d), proj.reshape(d, 1))
    node_mask_out, edge_mask_out = _stage3(
        scores, mk, mi, edge_mask.astype(jnp.float32).reshape(1, e), n, k)
    return gated, node_mask_out, edge_mask_out
